# Initial kernel scaffold; baseline (speedup 1.0000x reference)
import jax
import jax.numpy as jnp
from jax.experimental import pallas as pl


def kernel(occ, prc, adj, W1, att_src1, att_dst1, b1, W2, att_src2, att_dst2, b2, Wd, bd):
    raise NotImplementedError("write your pallas kernel here")


# speedup vs baseline: 2118.6644x; 2118.6644x over previous
import numpy as np

import jax
import jax.numpy as jnp
from jax.experimental import pallas as pl

_B = 2
_NODES = 1024
_SEQ = 1024
_FOUR = _SEQ // 2 + 1
_FPAD = 640
_NEG = 0.2
_NFIX = 2 * _FOUR
_NPAD = 1152
_BR = 256
_HI = jax.lax.Precision.HIGHEST


def _dg(a, b, dims):
    return jax.lax.dot_general(a, b, (dims, ((), ())), precision=_HI,
                               preferred_element_type=jnp.float32)


def _dense_body(occ_ref, prc_ref, c_ref, w1_ref, b1_ref, w2_ref, b2_ref,
                wd_ref, out_ref):
    fo = _dg(occ_ref[:, :], c_ref[:, :], (((1,), (0,))))
    fp = _dg(prc_ref[:, :], c_ref[:, :], (((1,), (0,))))
    acc = jnp.zeros_like(fo)
    for k in range(30):
        h = fo * w1_ref[0, k] + fp * w1_ref[1, k] + b1_ref[0, k]
        g = jnp.where(h > 0, h, jnp.exp(h) - 1.0)
        acc = acc + g * w2_ref[k, 0]
    h2 = acc + b2_ref[0, 0]
    out_ref[:, :] = _dg(h2, wd_ref[:, :], (((1,), (0,))))


def _masked_softmax_cols(sc, dc, cnt):
    n = sc.shape[0]
    ch = dc.shape[0]
    ones_n = jnp.ones((n, 1), jnp.float32)
    ones_c = jnp.ones((ch, 1), jnp.float32)
    m = _dg(sc, ones_c, (((1,), (1,)))) + _dg(ones_n, dc, (((1,), (1,))))
    l = jnp.where(m > 0, m, _NEG * m)
    lc = jnp.where(cnt > 0, l, -1e30)
    amax = jnp.max(lc, axis=0, keepdims=True)
    e = cnt * jnp.exp(lc - amax)
    den = jnp.sum(e, axis=0, keepdims=True)
    return e / den


def _fix_body(occ01_ref, prc01_ref, cflat_ref, esel_ref, adjcnt_ref,
              w1_ref, as_ref, ad_ref, b1_ref, w2_ref, a2_ref, b2bd_ref,
              wd2_ref, out_ref):
    cflat = cflat_ref[:, :]
    xo2 = _dg(cflat, occ01_ref[:, :], (((1,), (1,))))
    xp2 = _dg(cflat, prc01_ref[:, :], (((1,), (1,))))
    es = esel_ref[:, :]
    xo = jnp.sum(xo2 * es, axis=1, keepdims=True)
    xp = jnp.sum(xp2 * es, axis=1, keepdims=True)
    h1 = xo * w1_ref[0:1, :] + xp * w1_ref[1:2, :]
    asrc = _dg(h1, as_ref[:, :], (((1,), (0,))))
    adst = _dg(h1, ad_ref[:, :], (((1,), (0,))))

    chunks = [(0, _NPAD // 2), (_NPAD // 2, _NPAD)]
    out1_chunks = []
    for j0, j1 in chunks:
        cntb = adjcnt_ref[:, j0:j1]
        head_outs = []
        for hd in range(3):
            p = _masked_softmax_cols(asrc[:, hd:hd + 1],
                                     adst[j0:j1, hd:hd + 1], cntb)
            head_outs.append(_dg(p, h1[:, hd * 10:(hd + 1) * 10],
                                 (((0,), (0,)))))
        out1_chunks.append(jnp.concatenate(head_outs, axis=1))
    out1 = jnp.concatenate(out1_chunks, axis=0) + b1_ref[0:1, :]
    g = jnp.where(out1 > 0, out1, jnp.exp(out1) - 1.0)
    hh = _dg(g, w2_ref[:, :], (((1,), (0,))))
    as2 = hh * a2_ref[0, 0]
    ad2 = hh * a2_ref[0, 1]
    out2_chunks = []
    for j0, j1 in chunks:
        cntb = adjcnt_ref[:, j0:j1]
        p = _masked_softmax_cols(as2, ad2[j0:j1, :], cntb)
        out2_chunks.append(_dg(p, hh, (((0,), (0,)))))
    out2 = jnp.concatenate(out2_chunks, axis=0) + b2bd_ref[0, 0]
    res = _dg(out2, wd2_ref[:, :], (((0,), (0,))))
    out_ref[:, :] = res + b2bd_ref[0, 1]


def kernel(occ, prc, adj, W1, att_src1, att_dst1, b1, W2, att_src2, att_dst2,
           b2, Wd, bd):
    f32 = jnp.float32
    occ2 = occ.reshape(_B * _NODES, _SEQ)
    prc2 = prc.reshape(_B * _NODES, _SEQ)

    t_i = jnp.arange(_SEQ, dtype=jnp.int32)[:, None]
    f_i = jnp.arange(_FPAD, dtype=jnp.int32)[None, :]
    tf = (t_i * f_i) % _SEQ
    cmat = jnp.cos(tf.astype(f32) * f32(2.0 * np.pi / _SEQ))
    cmat = cmat * (f_i < _FOUR).astype(f32)

    wdp = jnp.zeros((_FPAD, 1), f32).at[:_FOUR, :].set(Wd)
    b1r = b1.reshape(1, 30)
    b2r = b2.reshape(1, 1)

    out_dense = pl.pallas_call(
        _dense_body,
        grid=(_B * _NODES // _BR,),
        in_specs=[
            pl.BlockSpec((_BR, _SEQ), lambda i: (i, 0)),
            pl.BlockSpec((_BR, _SEQ), lambda i: (i, 0)),
            pl.BlockSpec((_SEQ, _FPAD), lambda i: (0, 0)),
            pl.BlockSpec((2, 30), lambda i: (0, 0)),
            pl.BlockSpec((1, 30), lambda i: (0, 0)),
            pl.BlockSpec((30, 1), lambda i: (0, 0)),
            pl.BlockSpec((1, 1), lambda i: (0, 0)),
            pl.BlockSpec((_FPAD, 1), lambda i: (0, 0)),
        ],
        out_specs=pl.BlockSpec((_BR, 1), lambda i: (i, 0)),
        out_shape=jax.ShapeDtypeStruct((_B * _NODES, 1), f32),
    )(occ2, prc2, cmat, W1, b1r, W2, b2r, wdp)

    occ01 = occ[0, 0:2, :]
    prc01 = prc[0, 0:2, :]
    r = jnp.arange(_NPAD, dtype=jnp.int32)
    nr = r // _FOUR
    fr = r % _FOUR
    valid = r < _NFIX
    tfr = (jnp.arange(_SEQ, dtype=jnp.int32)[None, :] * fr[:, None]) % _SEQ
    cflat = jnp.cos(tfr.astype(f32) * f32(2.0 * np.pi / _SEQ))
    esel = jnp.stack([(nr == 0) & valid, (nr == 1) & valid],
                     axis=1).astype(f32)
    adjcnt = (jnp.zeros((_NPAD, _NPAD), f32).at[:_NODES, :_NODES].set(adj)
              + jnp.eye(_NPAD, dtype=f32))
    a_s = (att_src1[:, :, None] * jnp.eye(3, dtype=f32)[:, None, :]
           ).reshape(30, 3)
    a_d = (att_dst1[:, :, None] * jnp.eye(3, dtype=f32)[:, None, :]
           ).reshape(30, 3)
    a2 = jnp.concatenate([att_src2.reshape(1, 1), att_dst2.reshape(1, 1)],
                         axis=1)
    b2bd = jnp.concatenate([b2.reshape(1, 1), bd.reshape(1, 1)], axis=1)
    wd2 = (jnp.zeros((_NPAD, 2), f32)
           .at[0:_FOUR, 0].set(Wd[:, 0])
           .at[_FOUR:_NFIX, 1].set(Wd[:, 0]))

    fix = pl.pallas_call(
        _fix_body,
        out_shape=jax.ShapeDtypeStruct((1, 2), f32),
    )(occ01, prc01, cflat, esel, adjcnt, W1, a_s, a_d, b1r, W2, a2, b2bd, wd2)

    out = out_dense.reshape(_B, _NODES, 1)
    out = out.at[0, 0:2, 0].set(fix[0])
    return out


# speedup vs baseline: 3421.9955x; 1.6152x over previous
import numpy as np

import jax
import jax.numpy as jnp
from jax.experimental import pallas as pl

_B = 2
_NODES = 1024
_SEQ = 1024
_FOUR = _SEQ // 2 + 1
_FPAD = 640
_NEG = 0.2
_NFIX = 2 * _FOUR
_NPAD = 1152
_BR = 256


def _dg(a, b, dims):
    return jax.lax.dot_general(a, b, (dims, ((), ())),
                               preferred_element_type=jnp.float32)


def _dense_body(occ_ref, prc_ref, c_ref, w1_ref, b1_ref, w2_ref, b2_ref,
                wd_ref, out_ref):
    fo = _dg(occ_ref[:, :], c_ref[:, :], (((1,), (0,))))
    fp = _dg(prc_ref[:, :], c_ref[:, :], (((1,), (0,))))
    acc = jnp.zeros_like(fo)
    for k in range(30):
        h = fo * w1_ref[0, k] + fp * w1_ref[1, k] + b1_ref[0, k]
        g = jnp.where(h > 0, h, jnp.exp(h) - 1.0)
        acc = acc + g * w2_ref[k, 0]
    h2 = acc + b2_ref[0, 0]
    out_ref[:, :] = _dg(h2, wd_ref[:, :], (((1,), (0,))))


def _masked_softmax_cols(sc, dc, cnt):
    n = sc.shape[0]
    ch = dc.shape[0]
    ones_n = jnp.ones((n, 1), jnp.float32)
    ones_c = jnp.ones((ch, 1), jnp.float32)
    m = _dg(sc, ones_c, (((1,), (1,)))) + _dg(ones_n, dc, (((1,), (1,))))
    l = jnp.where(m > 0, m, _NEG * m)
    lc = jnp.where(cnt > 0, l, -1e30)
    amax = jnp.max(lc, axis=0, keepdims=True)
    e = cnt * jnp.exp(lc - amax)
    den = jnp.sum(e, axis=0, keepdims=True)
    return e / den


def _fix_body(occ01_ref, prc01_ref, cflat_ref, esel_ref, adjcnt_ref,
              w1_ref, as_ref, ad_ref, b1_ref, w2_ref, a2_ref, b2bd_ref,
              wd2_ref, out_ref):
    cflat = cflat_ref[:, :]
    xo2 = _dg(cflat, occ01_ref[:, :], (((1,), (1,))))
    xp2 = _dg(cflat, prc01_ref[:, :], (((1,), (1,))))
    es = esel_ref[:, :]
    xo = jnp.sum(xo2 * es, axis=1, keepdims=True)
    xp = jnp.sum(xp2 * es, axis=1, keepdims=True)
    h1 = xo * w1_ref[0:1, :] + xp * w1_ref[1:2, :]
    asrc = _dg(h1, as_ref[:, :], (((1,), (0,))))
    adst = _dg(h1, ad_ref[:, :], (((1,), (0,))))

    chunks = [(0, _NPAD // 2), (_NPAD // 2, _NPAD)]
    out1_chunks = []
    for j0, j1 in chunks:
        cntb = adjcnt_ref[:, j0:j1]
        head_outs = []
        for hd in range(3):
            p = _masked_softmax_cols(asrc[:, hd:hd + 1],
                                     adst[j0:j1, hd:hd + 1], cntb)
            head_outs.append(_dg(p, h1[:, hd * 10:(hd + 1) * 10],
                                 (((0,), (0,)))))
        out1_chunks.append(jnp.concatenate(head_outs, axis=1))
    out1 = jnp.concatenate(out1_chunks, axis=0) + b1_ref[0:1, :]
    g = jnp.where(out1 > 0, out1, jnp.exp(out1) - 1.0)
    hh = _dg(g, w2_ref[:, :], (((1,), (0,))))
    as2 = hh * a2_ref[0, 0]
    ad2 = hh * a2_ref[0, 1]
    out2_chunks = []
    for j0, j1 in chunks:
        cntb = adjcnt_ref[:, j0:j1]
        p = _masked_softmax_cols(as2, ad2[j0:j1, :], cntb)
        out2_chunks.append(_dg(p, hh, (((0,), (0,)))))
    out2 = jnp.concatenate(out2_chunks, axis=0) + b2bd_ref[0, 0]
    res = _dg(out2, wd2_ref[:, :], (((0,), (0,))))
    out_ref[:, :] = res + b2bd_ref[0, 1]


def kernel(occ, prc, adj, W1, att_src1, att_dst1, b1, W2, att_src2, att_dst2,
           b2, Wd, bd):
    f32 = jnp.float32
    occ2 = occ.reshape(_B * _NODES, _SEQ)
    prc2 = prc.reshape(_B * _NODES, _SEQ)

    t_i = jnp.arange(_SEQ, dtype=jnp.int32)[:, None]
    f_i = jnp.arange(_FPAD, dtype=jnp.int32)[None, :]
    tf = (t_i * f_i) % _SEQ
    cmat = jnp.cos(tf.astype(f32) * f32(2.0 * np.pi / _SEQ))
    cmat = cmat * (f_i < _FOUR).astype(f32)

    wdp = jnp.zeros((_FPAD, 1), f32).at[:_FOUR, :].set(Wd)
    b1r = b1.reshape(1, 30)
    b2r = b2.reshape(1, 1)

    out_dense = pl.pallas_call(
        _dense_body,
        grid=(_B * _NODES // _BR,),
        in_specs=[
            pl.BlockSpec((_BR, _SEQ), lambda i: (i, 0)),
            pl.BlockSpec((_BR, _SEQ), lambda i: (i, 0)),
            pl.BlockSpec((_SEQ, _FPAD), lambda i: (0, 0)),
            pl.BlockSpec((2, 30), lambda i: (0, 0)),
            pl.BlockSpec((1, 30), lambda i: (0, 0)),
            pl.BlockSpec((30, 1), lambda i: (0, 0)),
            pl.BlockSpec((1, 1), lambda i: (0, 0)),
            pl.BlockSpec((_FPAD, 1), lambda i: (0, 0)),
        ],
        out_specs=pl.BlockSpec((_BR, 1), lambda i: (i, 0)),
        out_shape=jax.ShapeDtypeStruct((_B * _NODES, 1), f32),
    )(occ2, prc2, cmat, W1, b1r, W2, b2r, wdp)

    occ01 = occ[0, 0:2, :]
    prc01 = prc[0, 0:2, :]
    r = jnp.arange(_NPAD, dtype=jnp.int32)
    nr = r // _FOUR
    fr = r % _FOUR
    valid = r < _NFIX
    tfr = (jnp.arange(_SEQ, dtype=jnp.int32)[None, :] * fr[:, None]) % _SEQ
    cflat = jnp.cos(tfr.astype(f32) * f32(2.0 * np.pi / _SEQ))
    esel = jnp.stack([(nr == 0) & valid, (nr == 1) & valid],
                     axis=1).astype(f32)
    adjcnt = (jnp.zeros((_NPAD, _NPAD), f32).at[:_NODES, :_NODES].set(adj)
              + jnp.eye(_NPAD, dtype=f32))
    a_s = (att_src1[:, :, None] * jnp.eye(3, dtype=f32)[:, None, :]
           ).reshape(30, 3)
    a_d = (att_dst1[:, :, None] * jnp.eye(3, dtype=f32)[:, None, :]
           ).reshape(30, 3)
    a2 = jnp.concatenate([att_src2.reshape(1, 1), att_dst2.reshape(1, 1)],
                         axis=1)
    b2bd = jnp.concatenate([b2.reshape(1, 1), bd.reshape(1, 1)], axis=1)
    wd2 = (jnp.zeros((_NPAD, 2), f32)
           .at[0:_FOUR, 0].set(Wd[:, 0])
           .at[_FOUR:_NFIX, 1].set(Wd[:, 0]))

    fix = pl.pallas_call(
        _fix_body,
        out_shape=jax.ShapeDtypeStruct((1, 2), f32),
    )(occ01, prc01, cflat, esel, adjcnt, W1, a_s, a_d, b1r, W2, a2, b2bd, wd2)

    out = out_dense.reshape(_B, _NODES, 1)
    out = out.at[0, 0:2, 0].set(fix[0])
    return out


# speedup vs baseline: 4511.0942x; 1.3183x over previous
import numpy as np

import jax
import jax.numpy as jnp
from jax.experimental import pallas as pl

_B = 2
_NODES = 1024
_SEQ = 1024
_FOUR = _SEQ // 2 + 1
_FPAD = 640
_NEG = 0.2
_NFIX = 2 * _FOUR
_NPAD = 1152
_BR = 256

_T = np.arange(_SEQ, dtype=np.int64)[:, None]
_F = np.arange(_FPAD, dtype=np.int64)[None, :]
_CMAT = (np.cos(((_T * _F) % _SEQ).astype(np.float64) * (2.0 * np.pi / _SEQ))
         * (_F < _FOUR)).astype(np.float32)
_DCROW = (0.5 * _CMAT.astype(np.float64).sum(axis=0)).astype(
    np.float32).reshape(1, _FPAD)

_R = np.arange(_NPAD, dtype=np.int64)
_FR = (_R % _FOUR)[:, None]
_CFLAT = np.cos(((np.arange(_SEQ, dtype=np.int64)[None, :] * _FR) % _SEQ)
                .astype(np.float64) * (2.0 * np.pi / _SEQ)).astype(np.float32)
_NR = _R // _FOUR
_ESEL = np.stack([(_NR == 0) & (_R < _NFIX), (_NR == 1) & (_R < _NFIX)],
                 axis=1).astype(np.float32)
_EYE = np.eye(_NPAD, dtype=np.float32)


def _dg(a, b, dims):
    return jax.lax.dot_general(a, b, (dims, ((), ())),
                               preferred_element_type=jnp.float32)


def _dense_body(occ_ref, prc_ref, c_ref, dc_ref, w1_ref, b1_ref, w2_ref,
                b2_ref, wd_ref, out_ref):
    fo = _dg(occ_ref[:, :] - 0.5, c_ref[:, :], (((1,), (0,)))) + dc_ref[:, :]
    fp = _dg(prc_ref[:, :] - 0.5, c_ref[:, :], (((1,), (0,)))) + dc_ref[:, :]
    acc = jnp.zeros_like(fo)
    for k in range(30):
        h = fo * w1_ref[0, k] + fp * w1_ref[1, k] + b1_ref[0, k]
        g = jnp.where(h > 0, h, jnp.exp(h) - 1.0)
        acc = acc + g * w2_ref[k, 0]
    h2 = acc + b2_ref[0, 0]
    out_ref[:, :] = _dg(h2, wd_ref[:, :], (((1,), (0,))))


def _fix_body(occ01_ref, prc01_ref, cflat_ref, esel_ref, adjp_ref, eye_ref,
              w1_ref, w1t_ref, as_ref, ad_ref, b1_ref, w2_ref, a2_ref,
              b2bd_ref, wd2_ref, out_ref):
    cflat = cflat_ref[:, :]
    xo2 = _dg(cflat, occ01_ref[:, :], (((1,), (1,))))
    xp2 = _dg(cflat, prc01_ref[:, :], (((1,), (1,))))
    es = esel_ref[:, :]
    xo = jnp.sum(xo2 * es, axis=1, keepdims=True)
    xp = jnp.sum(xp2 * es, axis=1, keepdims=True)
    eye = eye_ref[:, :]
    xot = _dg(xo, eye, (((0,), (0,))))
    xpt = _dg(xp, eye, (((0,), (0,))))
    h1 = xo * w1_ref[0:1, :] + xp * w1_ref[1:2, :]
    h1t = w1t_ref[:, 0:1] * xot + w1t_ref[:, 1:2] * xpt
    asrc = _dg(h1, as_ref[:, :], (((1,), (0,))))
    adstt = _dg(ad_ref[:, :], h1t, (((0,), (0,))))
    cnt = adjp_ref[:, :] + eye
    ones_n = jnp.ones((_NPAD, 1), jnp.float32)

    chunks = [(0, _NPAD // 2), (_NPAD // 2, _NPAD)]
    h1e = [jnp.concatenate([h1[:, hd * 10:(hd + 1) * 10], ones_n], axis=1)
           for hd in range(3)]
    out1_chunks = []
    for j0, j1 in chunks:
        cntb = cnt[:, j0:j1]
        head_outs = []
        for hd in range(3):
            m = asrc[:, hd:hd + 1] + adstt[hd:hd + 1, j0:j1]
            l = jnp.where(m > 0, m, _NEG * m)
            lc = jnp.where(cntb > 0, l, -1e30)
            amax = jnp.max(lc, axis=0, keepdims=True)
            e = cntb * jnp.exp(lc - amax)
            agg = _dg(e, h1e[hd], (((0,), (0,))))
            head_outs.append(agg[:, 0:10] / agg[:, 10:11])
        out1_chunks.append(jnp.concatenate(head_outs, axis=1))
    out1 = jnp.concatenate(out1_chunks, axis=0) + b1_ref[0:1, :]
    g = jnp.where(out1 > 0, out1, jnp.exp(out1) - 1.0)
    hh = _dg(g, w2_ref[:, :], (((1,), (0,))))
    hht = _dg(hh, eye, (((0,), (0,))))
    as2 = hh * a2_ref[0, 0]
    ad2t = hht * a2_ref[0, 1]
    hhe = jnp.concatenate([hh, ones_n], axis=1)
    out2_chunks = []
    for j0, j1 in chunks:
        cntb = cnt[:, j0:j1]
        m = as2 + ad2t[:, j0:j1]
        l = jnp.where(m > 0, m, _NEG * m)
        lc = jnp.where(cntb > 0, l, -1e30)
        amax = jnp.max(lc, axis=0, keepdims=True)
        e = cntb * jnp.exp(lc - amax)
        agg = _dg(e, hhe, (((0,), (0,))))
        out2_chunks.append(agg[:, 0:1] / agg[:, 1:2])
    out2 = jnp.concatenate(out2_chunks, axis=0) + b2bd_ref[0, 0]
    res = _dg(out2, wd2_ref[:, :], (((0,), (0,))))
    out_ref[:, :] = res + b2bd_ref[0, 1]


def kernel(occ, prc, adj, W1, att_src1, att_dst1, b1, W2, att_src2, att_dst2,
           b2, Wd, bd):
    f32 = jnp.float32
    occ2 = occ.reshape(_B * _NODES, _SEQ)
    prc2 = prc.reshape(_B * _NODES, _SEQ)

    cmat = jnp.asarray(_CMAT)
    dcrow = jnp.asarray(_DCROW)
    wdp = jnp.zeros((_FPAD, 1), f32).at[:_FOUR, :].set(Wd)
    b1r = b1.reshape(1, 30)
    b2r = b2.reshape(1, 1)

    out_dense = pl.pallas_call(
        _dense_body,
        grid=(_B * _NODES // _BR,),
        in_specs=[
            pl.BlockSpec((_BR, _SEQ), lambda i: (i, 0)),
            pl.BlockSpec((_BR, _SEQ), lambda i: (i, 0)),
            pl.BlockSpec((_SEQ, _FPAD), lambda i: (0, 0)),
            pl.BlockSpec((1, _FPAD), lambda i: (0, 0)),
            pl.BlockSpec((2, 30), lambda i: (0, 0)),
            pl.BlockSpec((1, 30), lambda i: (0, 0)),
            pl.BlockSpec((30, 1), lambda i: (0, 0)),
            pl.BlockSpec((1, 1), lambda i: (0, 0)),
            pl.BlockSpec((_FPAD, 1), lambda i: (0, 0)),
        ],
        out_specs=pl.BlockSpec((_BR, 1), lambda i: (i, 0)),
        out_shape=jax.ShapeDtypeStruct((_B * _NODES, 1), f32),
    )(occ2, prc2, cmat, dcrow, W1, b1r, W2, b2r, wdp)

    occ01 = occ[0, 0:2, :]
    prc01 = prc[0, 0:2, :]
    adjp = jnp.pad(adj, ((0, _NPAD - _NODES), (0, _NPAD - _NODES)))
    a_s = (att_src1[:, :, None] * jnp.eye(3, dtype=f32)[:, None, :]
           ).reshape(30, 3)
    a_d = (att_dst1[:, :, None] * jnp.eye(3, dtype=f32)[:, None, :]
           ).reshape(30, 3)
    a2 = jnp.concatenate([att_src2.reshape(1, 1), att_dst2.reshape(1, 1)],
                         axis=1)
    b2bd = jnp.concatenate([b2.reshape(1, 1), bd.reshape(1, 1)], axis=1)
    wd2 = (jnp.zeros((_NPAD, 2), f32)
           .at[0:_FOUR, 0].set(Wd[:, 0])
           .at[_FOUR:_NFIX, 1].set(Wd[:, 0]))

    fix = pl.pallas_call(
        _fix_body,
        out_shape=jax.ShapeDtypeStruct((1, 2), f32),
    )(occ01, prc01, jnp.asarray(_CFLAT), jnp.asarray(_ESEL), adjp,
      jnp.asarray(_EYE), W1, W1.T, a_s, a_d, b1r, W2, a2, b2bd, wd2)

    out = out_dense.reshape(_B, _NODES, 1)
    out = out.at[0, 0:2, 0].set(fix[0])
    return out


# speedup vs baseline: 4937.9518x; 1.0946x over previous
import numpy as np

import jax
import jax.numpy as jnp
from jax.experimental import pallas as pl

_B = 2
_NODES = 1024
_SEQ = 1024
_FOUR = _SEQ // 2 + 1
_FPAD = 640
_NEG = 0.2
_NFIX = 2 * _FOUR
_NPAD = 1152
_BR = 256

_T = np.arange(_SEQ, dtype=np.int64)[:, None]
_F = np.arange(_FPAD, dtype=np.int64)[None, :]
_CMAT = (np.cos(((_T * _F) % _SEQ).astype(np.float64) * (2.0 * np.pi / _SEQ))
         * (_F < _FOUR)).astype(np.float32)
_DCROW = (0.5 * _CMAT.astype(np.float64).sum(axis=0)).astype(
    np.float32).reshape(1, _FPAD)

_R = np.arange(_NPAD, dtype=np.int64)
_FR = (_R % _FOUR)[:, None]
_CFLAT = np.cos(((np.arange(_SEQ, dtype=np.int64)[None, :] * _FR) % _SEQ)
                .astype(np.float64) * (2.0 * np.pi / _SEQ)).astype(np.float32)
_NR = _R // _FOUR
_ESEL = np.stack([(_NR == 0) & (_R < _NFIX), (_NR == 1) & (_R < _NFIX)],
                 axis=1).astype(np.float32)
_EYE = np.eye(_NPAD, dtype=np.float32)


def _dg(a, b, dims):
    return jax.lax.dot_general(a, b, (dims, ((), ())),
                               preferred_element_type=jnp.float32)


def _body(occ_ref, prc_ref, c_ref, dc_ref, w1_ref, att1s_ref, att1d_ref,
          w2_ref, smalls_ref, wd_ref, cflat_ref, esel_ref, adj_ref, eye_ref,
          wd2_ref, out_ref):
    fo = _dg(occ_ref[:, :] - 0.5, c_ref[:, :], (((1,), (0,)))) + dc_ref[:, :]
    fp = _dg(prc_ref[:, :] - 0.5, c_ref[:, :], (((1,), (0,)))) + dc_ref[:, :]
    acc = jnp.zeros_like(fo)
    for k in range(30):
        h = fo * w1_ref[0, k] + fp * w1_ref[1, k] + smalls_ref[0, k]
        g = jnp.where(h > 0, h, jnp.exp(h) - 1.0)
        acc = acc + g * w2_ref[k, 0]
    h2 = acc + smalls_ref[0, 30]
    out_ref[:, :] = _dg(h2, wd_ref[:, :], (((1,), (0,))))

    @pl.when(pl.program_id(0) == 0)
    def _fix():
        cflat = cflat_ref[:, :]
        xo2 = _dg(cflat, occ_ref[0:2, :], (((1,), (1,))))
        xp2 = _dg(cflat, prc_ref[0:2, :], (((1,), (1,))))
        es = esel_ref[:, :]
        xo = jnp.sum(xo2 * es, axis=1, keepdims=True)
        xp = jnp.sum(xp2 * es, axis=1, keepdims=True)
        xoxp = jnp.concatenate([xo, xp], axis=1)
        eye = eye_ref[:, :]
        xoxpt = _dg(xoxp, eye, (((0,), (0,))))
        h1 = _dg(xoxp, w1_ref[:, :], (((1,), (0,))))
        h1t = _dg(w1_ref[:, :], xoxpt, (((0,), (0,))))
        zr = jnp.zeros((_NODES, _NPAD - _NODES), jnp.float32)
        zb = jnp.zeros((_NPAD - _NODES, _NPAD), jnp.float32)
        cnt = jnp.concatenate(
            [jnp.concatenate([adj_ref[:, :], zr], axis=1), zb], axis=0) + eye
        ones_n = jnp.ones((_NPAD, 1), jnp.float32)

        chunks = [(0, _NPAD // 2), (_NPAD // 2, _NPAD)]
        hs = []
        for hd in range(3):
            h1h = h1[:, hd * 10:(hd + 1) * 10]
            hs.append((
                _dg(h1h, att1s_ref[hd:hd + 1, :], (((1,), (1,)))),
                _dg(att1d_ref[hd:hd + 1, :], h1t[hd * 10:(hd + 1) * 10, :],
                    (((1,), (0,)))),
                jnp.concatenate([h1h, ones_n], axis=1),
            ))
        out1_chunks = []
        for j0, j1 in chunks:
            cntb = cnt[:, j0:j1]
            head_outs = []
            for asrc_h, adstt_h, h1e in hs:
                m = asrc_h + adstt_h[:, j0:j1]
                l = jnp.where(m > 0, m, _NEG * m)
                lc = jnp.where(cntb > 0, l, -1e30)
                amax = jnp.max(lc, axis=0, keepdims=True)
                e = cntb * jnp.exp(lc - amax)
                agg = _dg(e, h1e, (((0,), (0,))))
                head_outs.append(agg[:, 0:10] / agg[:, 10:11])
            out1_chunks.append(jnp.concatenate(head_outs, axis=1))
        out1 = (jnp.concatenate(out1_chunks, axis=0)
                + smalls_ref[0:1, 0:30])
        g = jnp.where(out1 > 0, out1, jnp.exp(out1) - 1.0)
        hh = _dg(g, w2_ref[:, :], (((1,), (0,))))
        hht = _dg(hh, eye, (((0,), (0,))))
        as2 = hh * smalls_ref[0, 32]
        ad2t = hht * smalls_ref[0, 33]
        hhe = jnp.concatenate([hh, ones_n], axis=1)
        out2_chunks = []
        for j0, j1 in chunks:
            cntb = cnt[:, j0:j1]
            m = as2 + ad2t[:, j0:j1]
            l = jnp.where(m > 0, m, _NEG * m)
            lc = jnp.where(cntb > 0, l, -1e30)
            amax = jnp.max(lc, axis=0, keepdims=True)
            e = cntb * jnp.exp(lc - amax)
            agg = _dg(e, hhe, (((0,), (0,))))
            out2_chunks.append(agg[:, 0:1] / agg[:, 1:2])
        out2 = jnp.concatenate(out2_chunks, axis=0) + smalls_ref[0, 30]
        res2 = _dg(wd2_ref[:, :], out2, (((0,), (0,))))
        out_ref[0:2, :] = res2 + smalls_ref[0, 31]


def kernel(occ, prc, adj, W1, att_src1, att_dst1, b1, W2, att_src2, att_dst2,
           b2, Wd, bd):
    f32 = jnp.float32
    occ2 = occ.reshape(_B * _NODES, _SEQ)
    prc2 = prc.reshape(_B * _NODES, _SEQ)

    wdp = jnp.zeros((_FPAD, 1), f32).at[:_FOUR, :].set(Wd)
    smalls = jnp.concatenate(
        [b1, b2, bd, att_src2[0], att_dst2[0]]).reshape(1, 34)
    wd2 = (jnp.zeros((_NPAD, 2), f32)
           .at[0:_FOUR, 0].set(Wd[:, 0])
           .at[_FOUR:_NFIX, 1].set(Wd[:, 0]))

    const = lambda i: (0, 0)
    out = pl.pallas_call(
        _body,
        grid=(_B * _NODES // _BR,),
        in_specs=[
            pl.BlockSpec((_BR, _SEQ), lambda i: (i, 0)),
            pl.BlockSpec((_BR, _SEQ), lambda i: (i, 0)),
            pl.BlockSpec((_SEQ, _FPAD), const),
            pl.BlockSpec((1, _FPAD), const),
            pl.BlockSpec((2, 30), const),
            pl.BlockSpec((3, 10), const),
            pl.BlockSpec((3, 10), const),
            pl.BlockSpec((30, 1), const),
            pl.BlockSpec((1, 34), const),
            pl.BlockSpec((_FPAD, 1), const),
            pl.BlockSpec((_NPAD, _SEQ), const),
            pl.BlockSpec((_NPAD, 2), const),
            pl.BlockSpec((_NODES, _NODES), const),
            pl.BlockSpec((_NPAD, _NPAD), const),
            pl.BlockSpec((_NPAD, 2), const),
        ],
        out_specs=pl.BlockSpec((_BR, 1), lambda i: (i, 0)),
        out_shape=jax.ShapeDtypeStruct((_B * _NODES, 1), f32),
    )(occ2, prc2, jnp.asarray(_CMAT), jnp.asarray(_DCROW), W1, att_src1,
      att_dst1, W2, smalls, wdp, jnp.asarray(_CFLAT), jnp.asarray(_ESEL),
      adj, jnp.asarray(_EYE), wd2)

    return out.reshape(_B, _NODES, 1)
